# baseline (speedup 1.0000x reference)
import functools

import jax
import jax.numpy as jnp
from jax import lax
from jax.experimental import pallas as pl
from jax.experimental.pallas import tpu as pltpu
from jax.experimental.pallas import tpu_sc as plsc

_NC = 2
_NS = 16
_NW = _NC * _NS
_NBUF = 8


def _sc_embedding_gather(table, ids):
    b, h = ids.shape
    d = table.shape[1]
    b_per_w = b // _NW
    rows_per_w = b_per_w * h
    n_rounds = b_per_w // _NBUF
    assert b % _NW == 0 and b_per_w % _NBUF == 0
    mesh = plsc.VectorSubcoreMesh(core_axis_name="c", subcore_axis_name="s")

    @functools.partial(
        pl.kernel,
        mesh=mesh,
        out_type=jax.ShapeDtypeStruct((b, h, d), jnp.float32),
        scratch_types=(
            [pltpu.VMEM((b_per_w, h), jnp.int32)]
            + [pltpu.VMEM((h, d), jnp.float32) for _ in range(_NBUF)]
            + [pltpu.SemaphoreType.DMA for _ in range(2 * _NBUF)]
        ),
        compiler_params=pltpu.CompilerParams(use_tc_tiling_on_sc=False),
    )
    def k(table_hbm, idx_hbm, out_hbm, idx_v, *scratch):
        bufs = scratch[:_NBUF]
        sem_g = scratch[_NBUF:2 * _NBUF]
        sem_w = scratch[2 * _NBUF:]
        wid = lax.axis_index("s") * _NC + lax.axis_index("c")
        rbase = wid * b_per_w
        pltpu.sync_copy(idx_hbm.at[pl.ds(wid * b_per_w, b_per_w)], idx_v)

        def fire_gather(slot, i):
            pltpu.async_copy(table_hbm.at[idx_v.at[i]], bufs[slot], sem_g[slot])

        for slot in range(_NBUF):
            fire_gather(slot, slot)

        def round_body(g, carry):
            ibase = g * _NBUF
            for slot in range(_NBUF):
                pltpu.make_async_copy(
                    table_hbm.at[idx_v.at[ibase + slot]], bufs[slot], sem_g[slot]
                ).wait()
                pltpu.async_copy(
                    bufs[slot], out_hbm.at[rbase + ibase + slot], sem_w[slot]
                )
            for slot in range(_NBUF):
                pltpu.make_async_copy(
                    bufs[slot], out_hbm.at[rbase + ibase + slot], sem_w[slot]
                ).wait()

                @pl.when(g < n_rounds - 1)
                def _():
                    fire_gather(slot, ibase + _NBUF + slot)

            return carry

        lax.fori_loop(0, n_rounds, round_body, 0)

    return k(table, ids)


def kernel(token_ids, embedding_table):
    return _sc_embedding_gather(embedding_table, token_ids.astype(jnp.int32))
